# baseline (speedup 1.0000x reference)
import functools

import jax
import jax.numpy as jnp
from jax import lax
from jax.experimental import pallas as pl
from jax.experimental.pallas import tpu as pltpu
from jax.experimental.pallas import tpu_sc as plsc

N = 10000
E = 320000
D_IN = 128
H = 256
C = 10
F_RAW = 1197

NS = 16
LANES = 16
BLK = 64
EPW = 20480
NBLK = EPW // BLK
E_PAD = NS * EPW
NPAD = 10240
ROWS_PER_TILE = NPAD // NS
NBUF = 4
G = 32


@functools.cache
def _mesh():
    return plsc.VectorSubcoreMesh(core_axis_name="c", subcore_axis_name="s")


def _fill(ref, rows, width, value):
    @pl.loop(0, rows)
    def _(j):
        for f in range(0, width, LANES):
            ref[j, pl.ds(f, LANES)] = jnp.full((LANES,), value, jnp.float32)


def _set_iota(idn_ref, r0):
    for c in range(0, BLK, LANES):
        idn_ref[pl.ds(c, LANES)] = (jnp.full((LANES,), r0 + c, jnp.int32)
                                    + lax.iota(jnp.int32, LANES))


def _scale_rows(rows_ref, e_all, b):
    @plsc.parallel_loop(0, BLK // LANES, unroll=2)
    def _(g):
        ev = e_all[b, pl.ds(g * LANES, LANES)]
        for l in range(LANES):
            s = ev[l]
            j = g * LANES + l
            for f in range(0, D_IN, LANES):
                rows_ref[j, pl.ds(f, LANES)] = rows_ref[j, pl.ds(f, LANES)] * s


def _zero_shared(acc_sh, zeros_v, idn_v, sid):
    base = sid * ROWS_PER_TILE
    for b in range(ROWS_PER_TILE // BLK):
        _set_iota(idn_v, base + b * BLK)
        pltpu.sync_copy(zeros_v, acc_sh.at[idn_v])


def _agg_loop(h_view, acc_sh, srcd, dstd, ed, sid,
              idxc, dstc, ec, rows2, semg, sems):
    @pl.loop(0, NBLK // G)
    def _(c):
        pltpu.sync_copy(srcd.at[sid, pl.ds(c * G, G)], idxc)
        pltpu.sync_copy(dstd.at[sid, pl.ds(c * G, G)], dstc)
        pltpu.sync_copy(ed.at[sid, pl.ds(c * G, G)], ec)
        pltpu.async_copy(h_view.at[idxc.at[0]], rows2.at[0], semg.at[0])
        pltpu.async_copy(h_view.at[idxc.at[1]], rows2.at[1], semg.at[1])

        @pl.loop(0, G // NBUF)
        def _(gg):
            for j in range(NBUF):
                b = gg * NBUF + j
                jn = (j + 2) % NBUF

                @pl.when(b >= 2)
                def _():
                    pltpu.make_async_copy(rows2.at[jn],
                                          acc_sh.at[dstc.at[b]],
                                          sems.at[jn]).wait()

                @pl.when(b + 2 < G)
                def _():
                    pltpu.async_copy(h_view.at[idxc.at[b + 2]], rows2.at[jn],
                                     semg.at[jn])

                pltpu.make_async_copy(h_view.at[idxc.at[b]], rows2.at[j],
                                      semg.at[j]).wait()
                _scale_rows(rows2.at[j], ec, b)
                pltpu.async_copy(rows2.at[j], acc_sh.at[dstc.at[b]],
                                 sems.at[j], add=True)

        for k in ((G - 2) % NBUF, (G - 1) % NBUF):
            pltpu.make_async_copy(rows2.at[k], acc_sh.at[dstc.at[0]],
                                  sems.at[k]).wait()


def _writeout(acc_sh, bounce_v, idn_v, out_view, sid):
    base = sid * ROWS_PER_TILE
    for b in range(ROWS_PER_TILE // BLK):
        r0 = base + b * BLK
        _set_iota(idn_v, r0)
        pltpu.sync_copy(acc_sh.at[idn_v], bounce_v)
        pltpu.sync_copy(bounce_v, out_view.at[pl.ds(r0, BLK)])


_SC_SCRATCH = [
    pltpu.VMEM((G, BLK), jnp.int32),
    pltpu.VMEM((G, BLK), jnp.int32),
    pltpu.VMEM((G, BLK), jnp.float32),
    pltpu.VMEM((BLK,), jnp.int32),
    pltpu.VMEM((NBUF, BLK, D_IN), jnp.float32),
    pltpu.SemaphoreType.DMA((NBUF,)),
    pltpu.SemaphoreType.DMA((NBUF,)),
    pltpu.VMEM_SHARED((NPAD, D_IN), jnp.float32),
]


@functools.cache
def _sc_layer1_kernel():
    return pl.kernel(
        _sc_layer1_body,
        out_type=(
            jax.ShapeDtypeStruct((1, NPAD, D_IN), jnp.float32),
            jax.ShapeDtypeStruct((NPAD, D_IN), jnp.float32),
        ),
        mesh=_mesh(),
        scratch_types=list(_SC_SCRATCH),
    )


def _sc_layer1_body(x_hbm, srcd, dstd, ed, s1_hbm, cnt_hbm,
                    idxc, dstc, ec, idn_v, rows2, semg, sems, s_sh):
    cid = lax.axis_index("c")
    sid = lax.axis_index("s")

    @pl.when(cid == 0)
    def _():
        _fill(rows2.at[0], BLK, D_IN, 0.0)
        _zero_shared(s_sh, rows2.at[0], idn_v, sid)
        plsc.subcore_barrier()
        _agg_loop(x_hbm, s_sh, srcd, dstd, ed, sid,
                  idxc, dstc, ec, rows2, semg, sems)
        plsc.subcore_barrier()
        _writeout(s_sh, rows2.at[0], idn_v, s1_hbm.at[0], sid)

    @pl.when(cid == 1)
    def _():
        _fill(rows2.at[0], BLK, D_IN, 0.0)
        _zero_shared(s_sh, rows2.at[0], idn_v, sid)
        _fill(rows2.at[0], BLK, D_IN, 1.0)
        plsc.subcore_barrier()

        @pl.loop(0, NBLK // G)
        def _(c):
            pltpu.sync_copy(dstd.at[sid, pl.ds(c * G, G)], dstc)

            @pl.loop(0, G)
            def _(b):
                pltpu.sync_copy(rows2.at[0], s_sh.at[dstc.at[b]], add=True)

        plsc.subcore_barrier()
        _writeout(s_sh, rows2.at[1], idn_v, cnt_hbm, sid)


@functools.cache
def _sc_layer23_kernel():
    return pl.kernel(
        _sc_layer23_body,
        out_type=jax.ShapeDtypeStruct((2, NPAD, D_IN), jnp.float32),
        mesh=_mesh(),
        scratch_types=list(_SC_SCRATCH),
    )


def _sc_layer23_body(h_hbm, srcd, dstd, ed, s_hbm,
                     idxc, dstc, ec, idn_v, rows2, semg, sems, s_sh):
    cid = lax.axis_index("c")
    sid = lax.axis_index("s")
    _fill(rows2.at[0], BLK, D_IN, 0.0)
    _zero_shared(s_sh, rows2.at[0], idn_v, sid)
    plsc.subcore_barrier()
    _agg_loop(h_hbm.at[cid], s_sh, srcd, dstd, ed, sid,
              idxc, dstc, ec, rows2, semg, sems)
    plsc.subcore_barrier()
    _writeout(s_sh, rows2.at[0], idn_v, s_hbm.at[cid], sid)


BN_ROWS = 1024
N_BLOCKS = NPAD // BN_ROWS


def _tc_layer_body(residual, s_ref, cnt_ref, w1_ref, b1_ref, w2_ref, b2_ref,
                   g_ref, be_ref, hb_ref, hsum_ref, stats_ref):
    p = pl.program_id(0)
    i = pl.program_id(1)
    k = s_ref.shape[0]
    agg = jnp.concatenate([s_ref[j] for j in range(k)], axis=-1)
    c = jnp.maximum(cnt_ref[:, 0:1], 1.0)
    agg = agg / c
    h1 = jnp.maximum(jnp.dot(agg, w1_ref[...],
                             preferred_element_type=jnp.float32)
                     + b1_ref[...], 0.0)
    h2 = jnp.maximum(jnp.dot(h1, w2_ref[...],
                             preferred_element_type=jnp.float32)
                     + b2_ref[...], 0.0)

    rows = lax.broadcasted_iota(jnp.int32, (BN_ROWS, 1), 0) + i * BN_ROWS
    msk = (rows < N).astype(jnp.float32)

    @pl.when(p == 0)
    def _():
        @pl.when(i == 0)
        def _():
            stats_ref[...] = jnp.zeros_like(stats_ref)
        h2m = h2 * msk
        stats_ref[0:1] += jnp.sum(h2m, axis=0, keepdims=True)
        stats_ref[1:2] += jnp.sum(h2m * h2, axis=0, keepdims=True)

    @pl.when(p == 1)
    def _():
        mu = stats_ref[0:1] / N
        var = stats_ref[1:2] / N - mu * mu
        hb = (g_ref[...] * (h2 - mu) * lax.rsqrt(var + 1e-5) + be_ref[...])
        if residual:
            hb = hb + agg
        hb_ref[0] = hb[:, :D_IN]
        hb_ref[1] = hb[:, D_IN:]

        @pl.when(i == 0)
        def _():
            hsum_ref[...] = jnp.zeros_like(hsum_ref)
        hsum_ref[0:1] += jnp.sum(hb * msk, axis=0, keepdims=True)


def _tc_layer(s, cnt, w1, b1, w2, b2, g, be, residual):
    k = s.shape[0]
    fin = k * D_IN
    body = functools.partial(_tc_layer_body, residual)
    return pl.pallas_call(
        body,
        grid=(2, N_BLOCKS),
        in_specs=[
            pl.BlockSpec((k, BN_ROWS, D_IN), lambda p, i: (0, i, 0)),
            pl.BlockSpec((BN_ROWS, D_IN), lambda p, i: (i, 0)),
            pl.BlockSpec((fin, H), lambda p, i: (0, 0)),
            pl.BlockSpec((1, H), lambda p, i: (0, 0)),
            pl.BlockSpec((H, H), lambda p, i: (0, 0)),
            pl.BlockSpec((1, H), lambda p, i: (0, 0)),
            pl.BlockSpec((1, H), lambda p, i: (0, 0)),
            pl.BlockSpec((1, H), lambda p, i: (0, 0)),
        ],
        out_specs=[
            pl.BlockSpec((2, BN_ROWS, D_IN), lambda p, i: (0, i, 0)),
            pl.BlockSpec((8, H), lambda p, i: (0, 0)),
        ],
        out_shape=[
            jax.ShapeDtypeStruct((2, NPAD, D_IN), jnp.float32),
            jax.ShapeDtypeStruct((8, H), jnp.float32),
        ],
        scratch_shapes=[pltpu.VMEM((8, H), jnp.float32)],
    )(s, cnt, w1, b1, w2, b2, g, be)


F_PAD = 1280


def _tc_head_body(hsum_ref, fp_ref, wf_ref, bf_ref, wc_ref, bc_ref,
                  wo_ref, bo_ref, out_ref):
    h_agg = hsum_ref[0:8] / N
    fr = jnp.maximum(jnp.dot(fp_ref[...], wf_ref[...],
                             preferred_element_type=jnp.float32)
                     + bf_ref[...], 0.0)
    comb = jnp.concatenate([h_agg, fr], axis=-1)
    hc = jnp.maximum(jnp.dot(comb, wc_ref[...],
                             preferred_element_type=jnp.float32)
                     + bc_ref[...], 0.0)
    out_ref[...] = (jnp.dot(hc, wo_ref[...],
                            preferred_element_type=jnp.float32)
                    + bo_ref[...])


def _tc_head(hsum, fp, wf, bf, wc, bc, wo, bo):
    return pl.pallas_call(
        _tc_head_body,
        out_shape=jax.ShapeDtypeStruct((8, 128), jnp.float32),
    )(hsum, fp, wf, bf, wc, bc, wo, bo)


def kernel(x, e, edge_index, features, W1_1, b1_1, W2_1, b2_1, g_1, be_1,
           W1_2, b1_2, W2_2, b2_2, g_2, be_2, W1_3, b1_3, W2_3, b2_3,
           g_3, be_3, Wf, bf, Wc, bc, Wo, bo):
    pad = E_PAD - E
    pad_dst = N + (jnp.arange(pad, dtype=jnp.int32) % (NPAD - N))
    src = jnp.concatenate([edge_index[0],
                           jnp.zeros((pad,), jnp.int32)]).reshape(NS, NBLK, BLK)
    dst = jnp.concatenate([edge_index[1],
                           pad_dst]).reshape(NS, NBLK, BLK)
    ew = jnp.concatenate([e[:, 0],
                          jnp.zeros((pad,), jnp.float32)]).reshape(NS, NBLK, BLK)

    s1, cnt = _sc_layer1_kernel()(x, src, dst, ew)
    hb1, _ = _tc_layer(s1, cnt, W1_1, b1_1.reshape(1, H), W2_1,
                       b2_1.reshape(1, H), g_1.reshape(1, H),
                       be_1.reshape(1, H), residual=False)
    s2 = _sc_layer23_kernel()(hb1, src, dst, ew)
    hb2, _ = _tc_layer(s2, cnt, W1_2, b1_2.reshape(1, H), W2_2,
                       b2_2.reshape(1, H), g_2.reshape(1, H),
                       be_2.reshape(1, H), residual=True)
    s3 = _sc_layer23_kernel()(hb2, src, dst, ew)
    _, hsum = _tc_layer(s3, cnt, W1_3, b1_3.reshape(1, H), W2_3,
                        b2_3.reshape(1, H), g_3.reshape(1, H),
                        be_3.reshape(1, H), residual=True)

    fp = jnp.zeros((8, F_PAD), jnp.float32).at[0:1, :F_RAW].set(features)
    wf = jnp.zeros((F_PAD, H), jnp.float32).at[:F_RAW].set(Wf)
    wo = jnp.zeros((H, 128), jnp.float32).at[:, :C].set(Wo)
    bo_p = jnp.zeros((1, 128), jnp.float32).at[0, :C].set(bo)
    out = _tc_head(hsum, fp, wf, bf.reshape(1, H), Wc, bc.reshape(1, H),
                   wo, bo_p)
    return out[0:1, :C]


# speedup vs baseline: 1.0336x; 1.0336x over previous
import functools

import jax
import jax.numpy as jnp
from jax import lax
from jax.experimental import pallas as pl
from jax.experimental.pallas import tpu as pltpu
from jax.experimental.pallas import tpu_sc as plsc

N = 10000
E = 320000
D_IN = 128
H = 256
C = 10
F_RAW = 1197

NS = 16
LANES = 16
BLK = 128
EPW = 20480
NBLK = EPW // BLK
E_PAD = NS * EPW
NPAD = 10240
ROWS_PER_TILE = NPAD // NS
NBUF = 2
G = 32


@functools.cache
def _mesh():
    return plsc.VectorSubcoreMesh(core_axis_name="c", subcore_axis_name="s")


def _fill(ref, rows, width, value):
    @pl.loop(0, rows)
    def _(j):
        for f in range(0, width, LANES):
            ref[j, pl.ds(f, LANES)] = jnp.full((LANES,), value, jnp.float32)


def _set_iota(idn_ref, r0):
    for c in range(0, BLK, LANES):
        idn_ref[pl.ds(c, LANES)] = (jnp.full((LANES,), r0 + c, jnp.int32)
                                    + lax.iota(jnp.int32, LANES))


def _scale_rows(rows_ref, e_all, b):
    @plsc.parallel_loop(0, BLK // LANES, unroll=2)
    def _(g):
        ev = e_all[b, pl.ds(g * LANES, LANES)]
        for l in range(LANES):
            s = ev[l]
            j = g * LANES + l
            for f in range(0, D_IN, LANES):
                rows_ref[j, pl.ds(f, LANES)] = rows_ref[j, pl.ds(f, LANES)] * s


def _zero_shared(acc_sh, zeros_v, idn_v, sid):
    base = sid * ROWS_PER_TILE
    for b in range(ROWS_PER_TILE // BLK):
        _set_iota(idn_v, base + b * BLK)
        pltpu.sync_copy(zeros_v, acc_sh.at[idn_v])


def _agg_loop(h_view, acc_sh, srcd, dstd, ed, sid,
              idxc, dstc, ec, rows2, semg, sems):
    @pl.loop(0, NBLK // G)
    def _(c):
        pltpu.sync_copy(srcd.at[sid, pl.ds(c * G, G)], idxc)
        pltpu.sync_copy(dstd.at[sid, pl.ds(c * G, G)], dstc)
        pltpu.sync_copy(ed.at[sid, pl.ds(c * G, G)], ec)
        pltpu.async_copy(h_view.at[idxc.at[0]], rows2.at[0], semg.at[0])

        @pl.loop(0, G // 2)
        def _(gg):
            for j in range(2):
                b = gg * 2 + j
                jn = (j + 1) % 2

                @pl.when(b >= 1)
                def _():
                    pltpu.make_async_copy(rows2.at[jn],
                                          acc_sh.at[dstc.at[b]],
                                          sems.at[jn]).wait()

                @pl.when(b + 1 < G)
                def _():
                    pltpu.async_copy(h_view.at[idxc.at[b + 1]], rows2.at[jn],
                                     semg.at[jn])

                pltpu.make_async_copy(h_view.at[idxc.at[b]], rows2.at[j],
                                      semg.at[j]).wait()
                _scale_rows(rows2.at[j], ec, b)
                pltpu.async_copy(rows2.at[j], acc_sh.at[dstc.at[b]],
                                 sems.at[j], add=True)

        pltpu.make_async_copy(rows2.at[1], acc_sh.at[dstc.at[0]],
                              sems.at[1]).wait()


def _writeout(acc_sh, bounce_v, idn_v, out_view, sid):
    base = sid * ROWS_PER_TILE
    for b in range(ROWS_PER_TILE // BLK):
        r0 = base + b * BLK
        _set_iota(idn_v, r0)
        pltpu.sync_copy(acc_sh.at[idn_v], bounce_v)
        pltpu.sync_copy(bounce_v, out_view.at[pl.ds(r0, BLK)])


_SC_SCRATCH = [
    pltpu.VMEM((G, BLK), jnp.int32),
    pltpu.VMEM((G, BLK), jnp.int32),
    pltpu.VMEM((G, BLK), jnp.float32),
    pltpu.VMEM((BLK,), jnp.int32),
    pltpu.VMEM((NBUF, BLK, D_IN), jnp.float32),
    pltpu.SemaphoreType.DMA((NBUF,)),
    pltpu.SemaphoreType.DMA((NBUF,)),
    pltpu.VMEM_SHARED((NPAD, D_IN), jnp.float32),
]


@functools.cache
def _sc_layer1_kernel():
    return pl.kernel(
        _sc_layer1_body,
        out_type=(
            jax.ShapeDtypeStruct((1, NPAD, D_IN), jnp.float32),
            jax.ShapeDtypeStruct((NPAD, D_IN), jnp.float32),
        ),
        mesh=_mesh(),
        scratch_types=list(_SC_SCRATCH),
    )


def _sc_layer1_body(x_hbm, srcd, dstd, ed, s1_hbm, cnt_hbm,
                    idxc, dstc, ec, idn_v, rows2, semg, sems, s_sh):
    cid = lax.axis_index("c")
    sid = lax.axis_index("s")

    @pl.when(cid == 0)
    def _():
        _fill(rows2.at[0], BLK, D_IN, 0.0)
        _zero_shared(s_sh, rows2.at[0], idn_v, sid)
        plsc.subcore_barrier()
        _agg_loop(x_hbm, s_sh, srcd, dstd, ed, sid,
                  idxc, dstc, ec, rows2, semg, sems)
        plsc.subcore_barrier()
        _writeout(s_sh, rows2.at[0], idn_v, s1_hbm.at[0], sid)

    @pl.when(cid == 1)
    def _():
        _fill(rows2.at[0], BLK, D_IN, 0.0)
        _zero_shared(s_sh, rows2.at[0], idn_v, sid)
        _fill(rows2.at[0], BLK, D_IN, 1.0)
        plsc.subcore_barrier()

        @pl.loop(0, NBLK // G)
        def _(c):
            pltpu.sync_copy(dstd.at[sid, pl.ds(c * G, G)], dstc)

            @pl.loop(0, G)
            def _(b):
                pltpu.sync_copy(rows2.at[0], s_sh.at[dstc.at[b]], add=True)

        plsc.subcore_barrier()
        _writeout(s_sh, rows2.at[1], idn_v, cnt_hbm, sid)


@functools.cache
def _sc_layer23_kernel():
    return pl.kernel(
        _sc_layer23_body,
        out_type=jax.ShapeDtypeStruct((2, NPAD, D_IN), jnp.float32),
        mesh=_mesh(),
        scratch_types=list(_SC_SCRATCH),
    )


def _sc_layer23_body(h_hbm, srcd, dstd, ed, s_hbm,
                     idxc, dstc, ec, idn_v, rows2, semg, sems, s_sh):
    cid = lax.axis_index("c")
    sid = lax.axis_index("s")
    _fill(rows2.at[0], BLK, D_IN, 0.0)
    _zero_shared(s_sh, rows2.at[0], idn_v, sid)
    plsc.subcore_barrier()
    _agg_loop(h_hbm.at[cid], s_sh, srcd, dstd, ed, sid,
              idxc, dstc, ec, rows2, semg, sems)
    plsc.subcore_barrier()
    _writeout(s_sh, rows2.at[0], idn_v, s_hbm.at[cid], sid)


BN_ROWS = 1024
N_BLOCKS = NPAD // BN_ROWS


def _tc_layer_body(residual, s_ref, cnt_ref, w1_ref, b1_ref, w2_ref, b2_ref,
                   g_ref, be_ref, hb_ref, hsum_ref, stats_ref):
    p = pl.program_id(0)
    i = pl.program_id(1)
    k = s_ref.shape[0]
    agg = jnp.concatenate([s_ref[j] for j in range(k)], axis=-1)
    c = jnp.maximum(cnt_ref[:, 0:1], 1.0)
    agg = agg / c
    h1 = jnp.maximum(jnp.dot(agg, w1_ref[...],
                             preferred_element_type=jnp.float32)
                     + b1_ref[...], 0.0)
    h2 = jnp.maximum(jnp.dot(h1, w2_ref[...],
                             preferred_element_type=jnp.float32)
                     + b2_ref[...], 0.0)

    rows = lax.broadcasted_iota(jnp.int32, (BN_ROWS, 1), 0) + i * BN_ROWS
    msk = (rows < N).astype(jnp.float32)

    @pl.when(p == 0)
    def _():
        @pl.when(i == 0)
        def _():
            stats_ref[...] = jnp.zeros_like(stats_ref)
        h2m = h2 * msk
        stats_ref[0:1] += jnp.sum(h2m, axis=0, keepdims=True)
        stats_ref[1:2] += jnp.sum(h2m * h2, axis=0, keepdims=True)

    @pl.when(p == 1)
    def _():
        mu = stats_ref[0:1] / N
        var = stats_ref[1:2] / N - mu * mu
        hb = (g_ref[...] * (h2 - mu) * lax.rsqrt(var + 1e-5) + be_ref[...])
        if residual:
            hb = hb + agg
        hb_ref[0] = hb[:, :D_IN]
        hb_ref[1] = hb[:, D_IN:]

        @pl.when(i == 0)
        def _():
            hsum_ref[...] = jnp.zeros_like(hsum_ref)
        hsum_ref[0:1] += jnp.sum(hb * msk, axis=0, keepdims=True)


def _tc_layer(s, cnt, w1, b1, w2, b2, g, be, residual):
    k = s.shape[0]
    fin = k * D_IN
    body = functools.partial(_tc_layer_body, residual)
    return pl.pallas_call(
        body,
        grid=(2, N_BLOCKS),
        in_specs=[
            pl.BlockSpec((k, BN_ROWS, D_IN), lambda p, i: (0, i, 0)),
            pl.BlockSpec((BN_ROWS, D_IN), lambda p, i: (i, 0)),
            pl.BlockSpec((fin, H), lambda p, i: (0, 0)),
            pl.BlockSpec((1, H), lambda p, i: (0, 0)),
            pl.BlockSpec((H, H), lambda p, i: (0, 0)),
            pl.BlockSpec((1, H), lambda p, i: (0, 0)),
            pl.BlockSpec((1, H), lambda p, i: (0, 0)),
            pl.BlockSpec((1, H), lambda p, i: (0, 0)),
        ],
        out_specs=[
            pl.BlockSpec((2, BN_ROWS, D_IN), lambda p, i: (0, i, 0)),
            pl.BlockSpec((8, H), lambda p, i: (0, 0)),
        ],
        out_shape=[
            jax.ShapeDtypeStruct((2, NPAD, D_IN), jnp.float32),
            jax.ShapeDtypeStruct((8, H), jnp.float32),
        ],
        scratch_shapes=[pltpu.VMEM((8, H), jnp.float32)],
    )(s, cnt, w1, b1, w2, b2, g, be)


F_PAD = 1280


def _tc_head_body(hsum_ref, fp_ref, wf_ref, bf_ref, wc_ref, bc_ref,
                  wo_ref, bo_ref, out_ref):
    h_agg = hsum_ref[0:8] / N
    fr = jnp.maximum(jnp.dot(fp_ref[...], wf_ref[...],
                             preferred_element_type=jnp.float32)
                     + bf_ref[...], 0.0)
    comb = jnp.concatenate([h_agg, fr], axis=-1)
    hc = jnp.maximum(jnp.dot(comb, wc_ref[...],
                             preferred_element_type=jnp.float32)
                     + bc_ref[...], 0.0)
    out_ref[...] = (jnp.dot(hc, wo_ref[...],
                            preferred_element_type=jnp.float32)
                    + bo_ref[...])


def _tc_head(hsum, fp, wf, bf, wc, bc, wo, bo):
    return pl.pallas_call(
        _tc_head_body,
        out_shape=jax.ShapeDtypeStruct((8, 128), jnp.float32),
    )(hsum, fp, wf, bf, wc, bc, wo, bo)


def kernel(x, e, edge_index, features, W1_1, b1_1, W2_1, b2_1, g_1, be_1,
           W1_2, b1_2, W2_2, b2_2, g_2, be_2, W1_3, b1_3, W2_3, b2_3,
           g_3, be_3, Wf, bf, Wc, bc, Wo, bo):
    pad = E_PAD - E
    pad_dst = N + (jnp.arange(pad, dtype=jnp.int32) % (NPAD - N))
    src = jnp.concatenate([edge_index[0],
                           jnp.zeros((pad,), jnp.int32)]).reshape(NS, NBLK, BLK)
    dst = jnp.concatenate([edge_index[1],
                           pad_dst]).reshape(NS, NBLK, BLK)
    ew = jnp.concatenate([e[:, 0],
                          jnp.zeros((pad,), jnp.float32)]).reshape(NS, NBLK, BLK)

    s1, cnt = _sc_layer1_kernel()(x, src, dst, ew)
    hb1, _ = _tc_layer(s1, cnt, W1_1, b1_1.reshape(1, H), W2_1,
                       b2_1.reshape(1, H), g_1.reshape(1, H),
                       be_1.reshape(1, H), residual=False)
    s2 = _sc_layer23_kernel()(hb1, src, dst, ew)
    hb2, _ = _tc_layer(s2, cnt, W1_2, b1_2.reshape(1, H), W2_2,
                       b2_2.reshape(1, H), g_2.reshape(1, H),
                       be_2.reshape(1, H), residual=True)
    s3 = _sc_layer23_kernel()(hb2, src, dst, ew)
    _, hsum = _tc_layer(s3, cnt, W1_3, b1_3.reshape(1, H), W2_3,
                        b2_3.reshape(1, H), g_3.reshape(1, H),
                        be_3.reshape(1, H), residual=True)

    fp = jnp.zeros((8, F_PAD), jnp.float32).at[0:1, :F_RAW].set(features)
    wf = jnp.zeros((F_PAD, H), jnp.float32).at[:F_RAW].set(Wf)
    wo = jnp.zeros((H, 128), jnp.float32).at[:, :C].set(Wo)
    bo_p = jnp.zeros((1, 128), jnp.float32).at[0, :C].set(bo)
    out = _tc_head(hsum, fp, wf, bf.reshape(1, H), Wc, bc.reshape(1, H),
                   wo, bo_p)
    return out[0:1, :C]
